# baseline (speedup 1.0000x reference)
import functools

import jax
import jax.numpy as jnp
from jax import lax
from jax.experimental import pallas as pl
from jax.experimental.pallas import tpu as pltpu
from jax.experimental.pallas import tpu_sc as plsc

N = 10000
E = 320000
D = 128

NP = 10240
NC = 2
NS = 16
NW = NC * NS
EB = 128
NB = 2560
NBW = NB // NW
EPAD = NB * EB
RPT = NP // NS
DG = 128
CHB = 40
CHH = 2
NBH = NS * CHH * CHB

_mesh = plsc.VectorSubcoreMesh(core_axis_name="c", subcore_axis_name="s")


def _make_scatter(half):
    row_base = half * NBH

    @functools.partial(
        pl.kernel,
        mesh=_mesh,
        out_type=jax.ShapeDtypeStruct((NP, D), jnp.float32),
        scratch_types=[
            pltpu.VMEM((CHB, EB), jnp.int32),
            pltpu.VMEM((CHB, EB), jnp.int32),
            pltpu.VMEM((EB, D), jnp.float32),
            pltpu.VMEM((EB, D), jnp.float32),
            pltpu.VMEM_SHARED((NP, D), jnp.float32),
            pltpu.SemaphoreType.DMA,
            pltpu.SemaphoreType.DMA,
        ],
    )
    def _sc_scatter(g_hbm, src_hbm, dst_hbm, zeros_hbm, out_hbm,
                    sidx, didx, rows_a, rows_b, acc, sem_a, sem_b):
        c = lax.axis_index("c")
        s = lax.axis_index("s")

        def gath(j, buf, sem):
            return pltpu.make_async_copy(g_hbm.at[sidx.at[j]], buf, sem)

        @pl.when(c == 0)
        def _():
            pltpu.sync_copy(zeros_hbm.at[pl.ds(s * RPT, RPT)],
                            acc.at[pl.ds(s * RPT, RPT)])

        plsc.subcore_barrier()

        @pl.when(c == 0)
        def _():
            row0 = row_base + s * (CHH * CHB)

            def chunk(p, carry):
                pltpu.sync_copy(src_hbm.at[pl.ds(row0 + p * CHB, CHB)],
                                sidx)
                pltpu.sync_copy(dst_hbm.at[pl.ds(row0 + p * CHB, CHB)],
                                didx)
                gath(0, rows_a, sem_a).start()

                def body(i, c2):
                    gath(i * 2, rows_a, sem_a).wait()
                    gath(i * 2 + 1, rows_b, sem_b).start()
                    pltpu.sync_copy(rows_a, acc.at[didx.at[i * 2]],
                                    add=True)

                    @pl.when(i < CHB // 2 - 1)
                    def _():
                        gath(i * 2 + 2, rows_a, sem_a).start()

                    gath(i * 2 + 1, rows_b, sem_b).wait()
                    pltpu.sync_copy(rows_b, acc.at[didx.at[i * 2 + 1]],
                                    add=True)
                    return c2

                lax.fori_loop(0, CHB // 2, body, 0)
                return carry

            lax.fori_loop(0, CHH, chunk, 0)

        plsc.subcore_barrier()

        @pl.when(c == 0)
        def _():
            pltpu.sync_copy(acc.at[pl.ds(s * RPT, RPT)],
                            out_hbm.at[pl.ds(s * RPT, RPT)])

    return _sc_scatter


_sc_scatter_a = _make_scatter(0)
_sc_scatter_b = _make_scatter(1)


@functools.partial(
    pl.kernel,
    mesh=_mesh,
    out_type=jax.ShapeDtypeStruct((NC, NP, DG), jnp.float32),
    scratch_types=[
        pltpu.VMEM((NBW, EB), jnp.int32),
        pltpu.VMEM((EB, DG), jnp.float32),
        pltpu.VMEM_SHARED((NP, DG), jnp.float32),
        pltpu.SemaphoreType.DMA,
    ],
)
def _sc_degree(dst_hbm, zeros_hbm, ones_hbm, out_hbm, didx, ones_v, acc, sem):
    c = lax.axis_index("c")
    s = lax.axis_index("s")
    wid = s * NC + c
    pltpu.sync_copy(zeros_hbm.at[pl.ds(s * RPT, RPT)],
                    acc.at[pl.ds(s * RPT, RPT)])
    pltpu.sync_copy(dst_hbm.at[pl.ds(wid * NBW, NBW)], didx)
    pltpu.sync_copy(ones_hbm, ones_v)
    plsc.subcore_barrier()

    def body(i, carry):
        pltpu.async_copy(ones_v, acc.at[didx.at[i * 2]], sem, add=True).wait()
        pltpu.async_copy(ones_v, acc.at[didx.at[i * 2 + 1]], sem,
                         add=True).wait()
        return carry

    lax.fori_loop(0, NBW // 2, body, 0)
    plsc.subcore_barrier()
    pltpu.sync_copy(acc.at[pl.ds(s * RPT, RPT)],
                    out_hbm.at[c, pl.ds(s * RPT, RPT)])


_RB = 512
_GRID = NP // _RB


def _dis_block(degp):
    deg = degp[0, :, 0] + degp[1, :, 0] + 1.0
    return lax.rsqrt(deg)


def _tc_first_body(degp_ref, x_ref, w_ref, g_ref):
    dis = _dis_block(degp_ref[...])
    h = jnp.dot(x_ref[...], w_ref[...], preferred_element_type=jnp.float32)
    g_ref[...] = h * dis[:, None]


def _tc_mid_body(degp_ref, sa_ref, sb_ref, g_ref, w_ref, b_ref, gn_ref):
    dis = _dis_block(degp_ref[...])
    agg = (sa_ref[...] + sb_ref[...] + g_ref[...]) * dis[:, None] + b_ref[...]
    h = jnp.maximum(agg, 0.0)
    gn_ref[...] = jnp.dot(h, w_ref[...],
                          preferred_element_type=jnp.float32) * dis[:, None]


def _tc_last_body(degp_ref, sa_ref, sb_ref, g_ref, b_ref, out_ref):
    dis = _dis_block(degp_ref[...])
    out_ref[...] = ((sa_ref[...] + sb_ref[...] + g_ref[...]) * dis[:, None]
                    + b_ref[...])


_degp_spec = pl.BlockSpec((NC, _RB, DG), lambda i: (0, i, 0))
_rows_spec = pl.BlockSpec((_RB, D), lambda i: (i, 0))
_sp_spec = pl.BlockSpec((NC, _RB, D), lambda i: (0, i, 0))
_w_spec = pl.BlockSpec((D, D), lambda i: (0, 0))
_b_spec = pl.BlockSpec((1, D), lambda i: (0, 0))
_out_rows = jax.ShapeDtypeStruct((NP, D), jnp.float32)

_tc_first = pl.pallas_call(
    _tc_first_body, grid=(_GRID,),
    in_specs=[_degp_spec, _rows_spec, _w_spec],
    out_specs=_rows_spec, out_shape=_out_rows)

_tc_mid = pl.pallas_call(
    _tc_mid_body, grid=(_GRID,),
    in_specs=[_degp_spec, _rows_spec, _rows_spec, _rows_spec, _w_spec,
              _b_spec],
    out_specs=_rows_spec, out_shape=_out_rows)

_tc_last = pl.pallas_call(
    _tc_last_body, grid=(_GRID,),
    in_specs=[_degp_spec, _rows_spec, _rows_spec, _rows_spec, _b_spec],
    out_specs=_rows_spec, out_shape=_out_rows)


def kernel(x, edge_index, W1, b1, W2, b2, W3, b3):
    x_pad = jnp.zeros((NP, D), x.dtype).at[:N].set(x)
    pad = jnp.full((EPAD - E,), N, jnp.int32)
    src = jnp.concatenate([edge_index[0], pad]).reshape(NB, EB)
    dst = jnp.concatenate([edge_index[1], pad]).reshape(NB, EB)
    zeros_nd = jnp.zeros((NP, D), jnp.float32)
    zeros_ng = jnp.zeros((NP, DG), jnp.float32)
    ones_eb = jnp.ones((EB, DG), jnp.float32)
    b1r = b1.reshape(1, D)
    b2r = b2.reshape(1, D)
    b3r = b3.reshape(1, D)

    degp = _sc_degree(dst, zeros_ng, ones_eb)
    g1 = _tc_first(degp, x_pad, W1)
    s1a = _sc_scatter_a(g1, src, dst, zeros_nd)
    s1b = _sc_scatter_b(g1, src, dst, zeros_nd)
    g2 = _tc_mid(degp, s1a, s1b, g1, W2, b1r)
    s2a = _sc_scatter_a(g2, src, dst, zeros_nd)
    s2b = _sc_scatter_b(g2, src, dst, zeros_nd)
    g3 = _tc_mid(degp, s2a, s2b, g2, W3, b2r)
    s3a = _sc_scatter_a(g3, src, dst, zeros_nd)
    s3b = _sc_scatter_b(g3, src, dst, zeros_nd)
    out = _tc_last(degp, s3a, s3b, g3, b3r)
    return out[:N]


# speedup vs baseline: 3.7613x; 3.7613x over previous
import functools

import jax
import jax.numpy as jnp
from jax import lax
from jax.experimental import pallas as pl
from jax.experimental.pallas import tpu as pltpu
from jax.experimental.pallas import tpu_sc as plsc

N = 10000
E = 320000
D = 128

NP = 10240
NC = 2
NS = 16
NW = NC * NS
EB = 128
NB = 2560
NBW = NB // NW
EPAD = NB * EB
RPT = NP // NS
DG = 128
CHB = 40
NCH = NBW // CHB

_mesh = plsc.VectorSubcoreMesh(core_axis_name="c", subcore_axis_name="s")


@functools.partial(
    pl.kernel,
    mesh=_mesh,
    out_type=[jax.ShapeDtypeStruct((NP, D), jnp.float32),
              jax.ShapeDtypeStruct((NP, D), jnp.float32)],
    scratch_types=[
        pltpu.VMEM((CHB, EB), jnp.int32),
        pltpu.VMEM((CHB, EB), jnp.int32),
        pltpu.VMEM((EB, D), jnp.float32),
        pltpu.VMEM((EB, D), jnp.float32),
        pltpu.VMEM_SHARED((NP, D), jnp.float32),
        pltpu.SemaphoreType.DMA,
        pltpu.SemaphoreType.DMA,
    ],
)
def _sc_scatter(g_hbm, src_hbm, dst_hbm, zeros_hbm, out_a_hbm, out_b_hbm,
                sidx, didx, rows_a, rows_b, acc, sem_a, sem_b):
    c = lax.axis_index("c")
    s = lax.axis_index("s")
    wid = s * NC + c
    pltpu.sync_copy(zeros_hbm.at[pl.ds(s * RPT, RPT)],
                    acc.at[pl.ds(s * RPT, RPT)])
    plsc.subcore_barrier()

    def gath(j, buf, sem):
        return pltpu.make_async_copy(g_hbm.at[sidx.at[j]], buf, sem)

    row0 = wid * NBW

    def chunk(p, carry):
        pltpu.sync_copy(src_hbm.at[pl.ds(row0 + p * CHB, CHB)], sidx)
        pltpu.sync_copy(dst_hbm.at[pl.ds(row0 + p * CHB, CHB)], didx)
        gath(0, rows_a, sem_a).start()

        def body(i, c2):
            gath(i * 2, rows_a, sem_a).wait()
            gath(i * 2 + 1, rows_b, sem_b).start()
            pltpu.sync_copy(rows_a, acc.at[didx.at[i * 2]], add=True)

            @pl.when(i < CHB // 2 - 1)
            def _():
                gath(i * 2 + 2, rows_a, sem_a).start()

            gath(i * 2 + 1, rows_b, sem_b).wait()
            pltpu.sync_copy(rows_b, acc.at[didx.at[i * 2 + 1]], add=True)
            return c2

        lax.fori_loop(0, CHB // 2, body, 0)
        return carry

    lax.fori_loop(0, NCH, chunk, 0)
    plsc.subcore_barrier()

    @pl.when(c == 0)
    def _():
        pltpu.sync_copy(acc.at[pl.ds(s * RPT, RPT)],
                        out_a_hbm.at[pl.ds(s * RPT, RPT)])

    @pl.when(c == 1)
    def _():
        pltpu.sync_copy(acc.at[pl.ds(s * RPT, RPT)],
                        out_b_hbm.at[pl.ds(s * RPT, RPT)])


@functools.partial(
    pl.kernel,
    mesh=_mesh,
    out_type=jax.ShapeDtypeStruct((NC, NP, DG), jnp.float32),
    scratch_types=[
        pltpu.VMEM((NBW, EB), jnp.int32),
        pltpu.VMEM((EB, DG), jnp.float32),
        pltpu.VMEM_SHARED((NP, DG), jnp.float32),
        pltpu.SemaphoreType.DMA,
    ],
)
def _sc_degree(dst_hbm, zeros_hbm, ones_hbm, out_hbm, didx, ones_v, acc, sem):
    c = lax.axis_index("c")
    s = lax.axis_index("s")
    wid = s * NC + c
    pltpu.sync_copy(zeros_hbm.at[pl.ds(s * RPT, RPT)],
                    acc.at[pl.ds(s * RPT, RPT)])
    pltpu.sync_copy(dst_hbm.at[pl.ds(wid * NBW, NBW)], didx)
    pltpu.sync_copy(ones_hbm, ones_v)
    plsc.subcore_barrier()

    def body(i, carry):
        pltpu.async_copy(ones_v, acc.at[didx.at[i * 2]], sem, add=True).wait()
        pltpu.async_copy(ones_v, acc.at[didx.at[i * 2 + 1]], sem,
                         add=True).wait()
        return carry

    lax.fori_loop(0, NBW // 2, body, 0)
    plsc.subcore_barrier()
    pltpu.sync_copy(acc.at[pl.ds(s * RPT, RPT)],
                    out_hbm.at[c, pl.ds(s * RPT, RPT)])


_RB = 512
_GRID = NP // _RB


def _dis_block(degp):
    deg = degp[0, :, 0] + degp[1, :, 0] + 1.0
    return lax.rsqrt(deg)


def _tc_first_body(degp_ref, x_ref, w_ref, g_ref):
    dis = _dis_block(degp_ref[...])
    h = jnp.dot(x_ref[...], w_ref[...], preferred_element_type=jnp.float32)
    g_ref[...] = h * dis[:, None]


def _tc_mid_body(degp_ref, sa_ref, sb_ref, g_ref, w_ref, b_ref, gn_ref):
    dis = _dis_block(degp_ref[...])
    agg = (sa_ref[...] + sb_ref[...] + g_ref[...]) * dis[:, None] + b_ref[...]
    h = jnp.maximum(agg, 0.0)
    gn_ref[...] = jnp.dot(h, w_ref[...],
                          preferred_element_type=jnp.float32) * dis[:, None]


def _tc_last_body(degp_ref, sa_ref, sb_ref, g_ref, b_ref, out_ref):
    dis = _dis_block(degp_ref[...])
    out_ref[...] = ((sa_ref[...] + sb_ref[...] + g_ref[...]) * dis[:, None]
                    + b_ref[...])


_degp_spec = pl.BlockSpec((NC, _RB, DG), lambda i: (0, i, 0))
_rows_spec = pl.BlockSpec((_RB, D), lambda i: (i, 0))
_sp_spec = pl.BlockSpec((NC, _RB, D), lambda i: (0, i, 0))
_w_spec = pl.BlockSpec((D, D), lambda i: (0, 0))
_b_spec = pl.BlockSpec((1, D), lambda i: (0, 0))
_out_rows = jax.ShapeDtypeStruct((NP, D), jnp.float32)

_tc_first = pl.pallas_call(
    _tc_first_body, grid=(_GRID,),
    in_specs=[_degp_spec, _rows_spec, _w_spec],
    out_specs=_rows_spec, out_shape=_out_rows)

_tc_mid = pl.pallas_call(
    _tc_mid_body, grid=(_GRID,),
    in_specs=[_degp_spec, _rows_spec, _rows_spec, _rows_spec, _w_spec,
              _b_spec],
    out_specs=_rows_spec, out_shape=_out_rows)

_tc_last = pl.pallas_call(
    _tc_last_body, grid=(_GRID,),
    in_specs=[_degp_spec, _rows_spec, _rows_spec, _rows_spec, _b_spec],
    out_specs=_rows_spec, out_shape=_out_rows)


def kernel(x, edge_index, W1, b1, W2, b2, W3, b3):
    x_pad = jnp.zeros((NP, D), x.dtype).at[:N].set(x)
    pad = N + (jnp.arange(EPAD - E, dtype=jnp.int32) % EB)
    src = jnp.concatenate([edge_index[0], pad]).reshape(NB, EB)
    dst = jnp.concatenate([edge_index[1], pad]).reshape(NB, EB)
    zeros_nd = jnp.zeros((NP, D), jnp.float32)
    zeros_ng = jnp.zeros((NP, DG), jnp.float32)
    ones_eb = jnp.ones((EB, DG), jnp.float32)
    b1r = b1.reshape(1, D)
    b2r = b2.reshape(1, D)
    b3r = b3.reshape(1, D)

    degp = _sc_degree(dst, zeros_ng, ones_eb)
    g1 = _tc_first(degp, x_pad, W1)
    s1a, s1b = _sc_scatter(g1, src, dst, zeros_nd)
    g2 = _tc_mid(degp, s1a, s1b, g1, W2, b1r)
    s2a, s2b = _sc_scatter(g2, src, dst, zeros_nd)
    g3 = _tc_mid(degp, s2a, s2b, g2, W3, b2r)
    s3a, s3b = _sc_scatter(g3, src, dst, zeros_nd)
    out = _tc_last(degp, s3a, s3b, g3, b3r)
    return out[:N]


# speedup vs baseline: 3.7789x; 1.0047x over previous
import functools

import jax
import jax.numpy as jnp
from jax import lax
from jax.experimental import pallas as pl
from jax.experimental.pallas import tpu as pltpu
from jax.experimental.pallas import tpu_sc as plsc

N = 10000
E = 320000
D = 128

NP = 10240
NC = 2
NS = 16
NW = NC * NS
EB = 128
NB = 2560
NBW = NB // NW
EPAD = NB * EB
RPT = NP // NS
DG = 128
CHB = 40
NCH = NBW // CHB

_mesh = plsc.VectorSubcoreMesh(core_axis_name="c", subcore_axis_name="s")


@functools.partial(
    pl.kernel,
    mesh=_mesh,
    out_type=[jax.ShapeDtypeStruct((NP, D), jnp.float32),
              jax.ShapeDtypeStruct((NP, D), jnp.float32)],
    scratch_types=[
        pltpu.VMEM((CHB, EB), jnp.int32),
        pltpu.VMEM((CHB, EB), jnp.int32),
        pltpu.VMEM((EB, D), jnp.float32),
        pltpu.VMEM((EB, D), jnp.float32),
        pltpu.VMEM_SHARED((NP, D), jnp.float32),
        pltpu.SemaphoreType.DMA,
        pltpu.SemaphoreType.DMA,
    ],
)
def _sc_scatter(g_hbm, src_hbm, dst_hbm, zeros_hbm, out_a_hbm, out_b_hbm,
                sidx, didx, rows_a, rows_b, acc, sem_a, sem_b):
    c = lax.axis_index("c")
    s = lax.axis_index("s")
    wid = s * NC + c
    pltpu.sync_copy(zeros_hbm.at[pl.ds(s * RPT, RPT)],
                    acc.at[pl.ds(s * RPT, RPT)])
    plsc.subcore_barrier()

    def gath(j, buf, sem):
        return pltpu.make_async_copy(g_hbm.at[sidx.at[j]], buf, sem)

    row0 = wid * NBW

    def chunk(p, carry):
        pltpu.sync_copy(src_hbm.at[pl.ds(row0 + p * CHB, CHB)], sidx)
        pltpu.sync_copy(dst_hbm.at[pl.ds(row0 + p * CHB, CHB)], didx)
        gath(0, rows_a, sem_a).start()

        def body(i, c2):
            gath(i * 2, rows_a, sem_a).wait()
            gath(i * 2 + 1, rows_b, sem_b).start()
            pltpu.sync_copy(rows_a, acc.at[didx.at[i * 2]], add=True)

            @pl.when(i < CHB // 2 - 1)
            def _():
                gath(i * 2 + 2, rows_a, sem_a).start()

            gath(i * 2 + 1, rows_b, sem_b).wait()
            pltpu.sync_copy(rows_b, acc.at[didx.at[i * 2 + 1]], add=True)
            return c2

        lax.fori_loop(0, CHB // 2, body, 0)
        return carry

    lax.fori_loop(0, NCH, chunk, 0)
    plsc.subcore_barrier()

    @pl.when(c == 0)
    def _():
        pltpu.sync_copy(acc.at[pl.ds(s * RPT, RPT)],
                        out_a_hbm.at[pl.ds(s * RPT, RPT)])

    @pl.when(c == 1)
    def _():
        pltpu.sync_copy(acc.at[pl.ds(s * RPT, RPT)],
                        out_b_hbm.at[pl.ds(s * RPT, RPT)])


@functools.partial(
    pl.kernel,
    mesh=_mesh,
    out_type=jax.ShapeDtypeStruct((NC, NP, D), jnp.float32),
    scratch_types=[
        pltpu.VMEM((NBW, EB), jnp.int32),
        pltpu.VMEM((EB, D), jnp.float32),
        pltpu.VMEM_SHARED((NP, D), jnp.float32),
    ],
)
def _sc_degree(dst_hbm, zeros_hbm, ones_hbm, out_hbm, didx, ones_v, acc):
    c = lax.axis_index("c")
    s = lax.axis_index("s")
    wid = s * NC + c
    pltpu.sync_copy(zeros_hbm.at[pl.ds(s * RPT, RPT)],
                    acc.at[pl.ds(s * RPT, RPT)])
    pltpu.sync_copy(dst_hbm.at[pl.ds(wid * NBW, NBW)], didx)
    pltpu.sync_copy(ones_hbm, ones_v)
    plsc.subcore_barrier()

    def body(i, carry):
        pltpu.sync_copy(ones_v, acc.at[didx.at[i * 2]], add=True)
        pltpu.sync_copy(ones_v, acc.at[didx.at[i * 2 + 1]], add=True)
        return carry

    lax.fori_loop(0, NBW // 2, body, 0)
    plsc.subcore_barrier()
    pltpu.sync_copy(acc.at[pl.ds(s * RPT, RPT)],
                    out_hbm.at[c, pl.ds(s * RPT, RPT)])


_RB = 512
_GRID = NP // _RB


_DW = 8


def _tc_first_body(degp_ref, x_ref, w_ref, g_ref, dis_ref):
    degp = degp_ref[...]
    dis = lax.rsqrt(degp[0, :, 0] + degp[1, :, 0] + 1.0)
    dis_ref[...] = jnp.broadcast_to(dis[:, None], (_RB, _DW))
    h = jnp.dot(x_ref[...], w_ref[...], preferred_element_type=jnp.float32)
    g_ref[...] = h * dis[:, None]


def _tc_mid_body(dis_ref, sa_ref, sb_ref, g_ref, w_ref, b_ref, gn_ref):
    dis = dis_ref[...][:, 0]
    agg = (sa_ref[...] + sb_ref[...] + g_ref[...]) * dis[:, None] + b_ref[...]
    h = jnp.maximum(agg, 0.0)
    gn_ref[...] = jnp.dot(h, w_ref[...],
                          preferred_element_type=jnp.float32) * dis[:, None]


def _tc_last_body(dis_ref, sa_ref, sb_ref, g_ref, b_ref, out_ref):
    dis = dis_ref[...][:, 0]
    out_ref[...] = ((sa_ref[...] + sb_ref[...] + g_ref[...]) * dis[:, None]
                    + b_ref[...])


_degp_spec = pl.BlockSpec((NC, _RB, D), lambda i: (0, i, 0))
_rows_spec = pl.BlockSpec((_RB, D), lambda i: (i, 0))
_dis_spec = pl.BlockSpec((_RB, _DW), lambda i: (i, 0))
_w_spec = pl.BlockSpec((D, D), lambda i: (0, 0))
_b_spec = pl.BlockSpec((1, D), lambda i: (0, 0))
_out_rows = jax.ShapeDtypeStruct((NP, D), jnp.float32)
_out_dis = jax.ShapeDtypeStruct((NP, _DW), jnp.float32)

_tc_first = pl.pallas_call(
    _tc_first_body, grid=(_GRID,),
    in_specs=[_degp_spec, _rows_spec, _w_spec],
    out_specs=[_rows_spec, _dis_spec], out_shape=[_out_rows, _out_dis])

_tc_mid = pl.pallas_call(
    _tc_mid_body, grid=(_GRID,),
    in_specs=[_dis_spec, _rows_spec, _rows_spec, _rows_spec, _w_spec,
              _b_spec],
    out_specs=_rows_spec, out_shape=_out_rows)

_tc_last = pl.pallas_call(
    _tc_last_body, grid=(_GRID,),
    in_specs=[_dis_spec, _rows_spec, _rows_spec, _rows_spec, _b_spec],
    out_specs=_rows_spec, out_shape=_out_rows)


def kernel(x, edge_index, W1, b1, W2, b2, W3, b3):
    x_pad = jnp.zeros((NP, D), x.dtype).at[:N].set(x)
    pad = N + (jnp.arange(EPAD - E, dtype=jnp.int32) % EB)
    src = jnp.concatenate([edge_index[0], pad]).reshape(NB, EB)
    dstf = jnp.concatenate([edge_index[1], pad])
    dst = dstf.reshape(NB, EB)
    zeros_nd = jnp.zeros((NP, D), jnp.float32)
    ones_ed = jnp.ones((EB, D), jnp.float32)
    b1r = b1.reshape(1, D)
    b2r = b2.reshape(1, D)
    b3r = b3.reshape(1, D)

    degp = _sc_degree(dst, zeros_nd, ones_ed)
    g1, dis2 = _tc_first(degp, x_pad, W1)
    s1a, s1b = _sc_scatter(g1, src, dst, zeros_nd)
    g2 = _tc_mid(dis2, s1a, s1b, g1, W2, b1r)
    s2a, s2b = _sc_scatter(g2, src, dst, zeros_nd)
    g3 = _tc_mid(dis2, s2a, s2b, g2, W3, b2r)
    s3a, s3b = _sc_scatter(g3, src, dst, zeros_nd)
    out = _tc_last(dis2, s3a, s3b, g3, b3r)
    return out[:N]
